# Initial kernel scaffold; baseline (speedup 1.0000x reference)
import jax
import jax.numpy as jnp
from jax.experimental import pallas as pl


def kernel(features, edge_index0, edge_index1, W_init, b_init, W_self, b_self, W_neigh, b_neigh):
    raise NotImplementedError("write your pallas kernel here")


# speedup vs baseline: 3.0879x; 3.0879x over previous
import functools

import jax
import jax.numpy as jnp
from jax import lax
from jax.experimental import pallas as pl
from jax.experimental.pallas import tpu as pltpu
from jax.experimental.pallas import tpu_sc as plsc

N = 10000
E = 320000
IN_DIM = 16
HID = 128

NC = 2
NS = 16
NW = NC * NS

CHUNK = 128
NCH = 80
EPT = NCH * CHUNK
E_PAD = NW * EPT
ACC_R = 10240
RPT = ACC_R // NS


def _sc_aggregate(h, src_r, dst_r):
    mesh = plsc.VectorSubcoreMesh(core_axis_name="c", subcore_axis_name="s")

    @functools.partial(
        pl.kernel,
        out_type=(
            jax.ShapeDtypeStruct((NC, N, HID), jnp.float32),
            jax.ShapeDtypeStruct((NW, 1, N), jnp.float32),
        ),
        mesh=mesh,
        scratch_types=(
            pltpu.VMEM((NCH, CHUNK), jnp.int32),
            pltpu.VMEM((NCH, CHUNK), jnp.int32),
            pltpu.VMEM((CHUNK, HID), jnp.float32),
            pltpu.VMEM((EPT,), jnp.float32),
            pltpu.VMEM_SHARED((ACC_R, HID), jnp.float32),
        ),
        compiler_params=pltpu.CompilerParams(needs_layout_passes=False),
    )
    def agg(h_hbm, src_hbm, dst_hbm, msum_out, deg_out,
            srcv, dstv, rows, degl, acc):
        cid = lax.axis_index("c")
        sid = lax.axis_index("s")
        wid = cid * NS + sid

        pltpu.sync_copy(src_hbm.at[wid], srcv)
        pltpu.sync_copy(dst_hbm.at[wid], dstv)

        def zero_deg(i, _):
            degl[pl.ds(i * 16, 16)] = jnp.zeros((16,), jnp.float32)
            return 0
        lax.fori_loop(0, EPT // 16, zero_deg, 0)

        def zero_rows(i, _):
            r = i // (HID // 16)
            c = i % (HID // 16)
            rows[r, pl.ds(c * 16, 16)] = jnp.zeros((16,), jnp.float32)
            return 0
        lax.fori_loop(0, CHUNK * (HID // 16), zero_rows, 0)

        zb = sid * RPT
        for t in range(RPT // CHUNK):
            pltpu.sync_copy(rows, acc.at[pl.ds(zb + t * CHUNK, CHUNK)])
        plsc.subcore_barrier()

        ones16 = jnp.ones((16,), jnp.float32)

        def chunk_body(j, _):
            pltpu.sync_copy(h_hbm.at[srcv.at[j]], rows)
            pltpu.sync_copy(rows, acc.at[dstv.at[j]], add=True)

            def deg_body(i, _):
                d16 = dstv[j, pl.ds(i * 16, 16)]
                plsc.addupdate_scatter(degl, [d16], ones16)
                return 0
            lax.fori_loop(0, CHUNK // 16, deg_body, 0)
            return 0

        lax.fori_loop(0, NCH, chunk_body, 0)
        plsc.subcore_barrier()

        ob = sid * 624
        pltpu.sync_copy(acc.at[pl.ds(ob, 624)],
                        msum_out.at[cid, pl.ds(ob, 624)])

        @pl.when(sid == NS - 1)
        def _():
            pltpu.sync_copy(acc.at[pl.ds(16 * 624, N - 16 * 624)],
                            msum_out.at[cid, pl.ds(16 * 624, N - 16 * 624)])

        pltpu.sync_copy(degl.at[pl.ds(0, N)], deg_out.at[wid, 0])

    return agg(h, src_r, dst_r)


def _tc_init(features, W_init, b_init):
    R = 2000

    def body(x_ref, w_ref, b_ref, o_ref):
        y = jnp.dot(x_ref[...], w_ref[...], preferred_element_type=jnp.float32)
        o_ref[...] = jnp.maximum(y + b_ref[...], 0.0)

    return pl.pallas_call(
        body,
        grid=(N // R,),
        in_specs=[
            pl.BlockSpec((R, IN_DIM), lambda i: (i, 0)),
            pl.BlockSpec((IN_DIM, HID), lambda i: (0, 0)),
            pl.BlockSpec((1, HID), lambda i: (0, 0)),
        ],
        out_specs=pl.BlockSpec((R, HID), lambda i: (i, 0)),
        out_shape=jax.ShapeDtypeStruct((N, HID), jnp.float32),
    )(features, W_init, b_init.reshape(1, HID))


def _tc_combine(h, msum_p, deg_t, W_self, b_self, W_neigh, b_neigh, act):
    R = 2000

    def body(h_ref, m_ref, d_ref, ws_ref, wn_ref, bs_ref, bn_ref, o_ref):
        h_blk = h_ref[...]
        msum = m_ref[0] + m_ref[1]
        deg = jnp.sum(d_ref[...], axis=1)
        h_neigh = msum / jnp.clip(deg, 1.0)[:, None]
        out = (jnp.dot(h_blk, ws_ref[...], preferred_element_type=jnp.float32)
               + jnp.dot(h_neigh, wn_ref[...], preferred_element_type=jnp.float32)
               + bs_ref[...] + bn_ref[...])
        if act:
            out = jnp.maximum(out, 0.0)
        o_ref[...] = out

    return pl.pallas_call(
        functools.partial(body),
        grid=(N // R,),
        in_specs=[
            pl.BlockSpec((R, HID), lambda i: (i, 0)),
            pl.BlockSpec((NC, R, HID), lambda i: (0, i, 0)),
            pl.BlockSpec((R, NW), lambda i: (i, 0)),
            pl.BlockSpec((HID, HID), lambda i: (0, 0)),
            pl.BlockSpec((HID, HID), lambda i: (0, 0)),
            pl.BlockSpec((1, HID), lambda i: (0, 0)),
            pl.BlockSpec((1, HID), lambda i: (0, 0)),
        ],
        out_specs=pl.BlockSpec((R, HID), lambda i: (i, 0)),
        out_shape=jax.ShapeDtypeStruct((N, HID), jnp.float32),
    )(h, msum_p, deg_t, W_self, W_neigh,
      b_self.reshape(1, HID), b_neigh.reshape(1, HID))


def _prep_edges(edge_index):
    src = edge_index[0].astype(jnp.int32)
    dst = edge_index[1].astype(jnp.int32)
    pad = E_PAD - E
    src_p = jnp.concatenate([src, jnp.zeros((pad,), jnp.int32)])
    dst_p = jnp.concatenate([dst, jnp.full((pad,), N, jnp.int32)])
    return src_p.reshape(NW, NCH, CHUNK), dst_p.reshape(NW, NCH, CHUNK)


def kernel(features, edge_index0, edge_index1, W_init, b_init,
           W_self, b_self, W_neigh, b_neigh):
    src0, dst0 = _prep_edges(edge_index0)
    src1, dst1 = _prep_edges(edge_index1)

    h = _tc_init(features, W_init, b_init)

    msum_p, deg_p = _sc_aggregate(h, src0, dst0)
    h = _tc_combine(h, msum_p, deg_p[:, 0, :].T, W_self, b_self, W_neigh,
                    b_neigh, act=True)

    msum_p, deg_p = _sc_aggregate(h, src1, dst1)
    h = _tc_combine(h, msum_p, deg_p[:, 0, :].T, W_self, b_self, W_neigh,
                    b_neigh, act=False)
    return h


# speedup vs baseline: 4.7643x; 1.5429x over previous
import functools

import jax
import jax.numpy as jnp
from jax import lax
from jax.experimental import pallas as pl
from jax.experimental.pallas import tpu as pltpu
from jax.experimental.pallas import tpu_sc as plsc

N = 10000
E = 320000
IN_DIM = 16
HID = 128
HH = HID // 2

NC = 2
NS = 16

CHUNK = 128
NCH = 160
EPT = NCH * CHUNK
E_PAD = NS * EPT
ACC_R = 10240
RPT = ACC_R // NS
NBUF = 4
G = 2


def _sc_aggregate(h_cat, src2, dst_r):
    mesh = plsc.VectorSubcoreMesh(core_axis_name="c", subcore_axis_name="s")

    @functools.partial(
        pl.kernel,
        out_type=(
            jax.ShapeDtypeStruct((NC, N, HH), jnp.float32),
            jax.ShapeDtypeStruct((NS, 1, N), jnp.float32),
        ),
        mesh=mesh,
        scratch_types=(
            pltpu.VMEM((NCH, CHUNK), jnp.int32),
            pltpu.VMEM((NCH, CHUNK), jnp.int32),
            pltpu.VMEM((NBUF, CHUNK, HH), jnp.float32),
            pltpu.VMEM((ACC_R,), jnp.float32),
            pltpu.VMEM_SHARED((ACC_R, HH), jnp.float32),
            pltpu.SemaphoreType.DMA,
            pltpu.SemaphoreType.DMA,
        ),
        compiler_params=pltpu.CompilerParams(needs_layout_passes=False,
                                             use_tc_tiling_on_sc=False),
    )
    def agg(h_hbm, src_hbm, dst_hbm, msum_out, deg_out,
            srcv, dstv, rows, degl, acc, gsem, ssem):
        cid = lax.axis_index("c")
        sid = lax.axis_index("s")

        pltpu.sync_copy(src_hbm.at[cid, sid], srcv)
        pltpu.sync_copy(dst_hbm.at[sid], dstv)

        def zero_deg(i, _):
            degl[pl.ds(i * 16, 16)] = jnp.zeros((16,), jnp.float32)
            return 0
        lax.fori_loop(0, ACC_R // 16, zero_deg, 0)

        def zero_rows(i, _):
            r = i // (HH // 16)
            c = i % (HH // 16)
            rows[0, r, pl.ds(c * 16, 16)] = jnp.zeros((16,), jnp.float32)
            return 0
        lax.fori_loop(0, CHUNK * (HH // 16), zero_rows, 0)

        zb = sid * RPT
        for t in range(RPT // CHUNK):
            pltpu.sync_copy(rows.at[0], acc.at[pl.ds(zb + t * CHUNK, CHUNK)])
        plsc.subcore_barrier()

        ones16 = jnp.ones((16,), jnp.float32)

        def g_fire(j):
            pltpu.async_copy(h_hbm.at[srcv.at[lax.rem(j, NCH)]],
                             rows.at[lax.rem(j, NBUF)], gsem)

        def g_wait(j):
            pltpu.make_async_copy(h_hbm.at[srcv.at[lax.rem(j, NCH)]],
                                  rows.at[lax.rem(j, NBUF)], gsem).wait()

        def s_fire(j):
            pltpu.async_copy(rows.at[lax.rem(j, NBUF)],
                             acc.at[dstv.at[lax.rem(j, NCH)]], ssem, add=True)

        def s_wait(j):
            pltpu.make_async_copy(rows.at[lax.rem(j, NBUF)],
                                  acc.at[dstv.at[lax.rem(j, NCH)]],
                                  ssem).wait()

        for j in range(G):
            g_fire(j)

        def chunk_body(j, _):
            g_wait(j)

            @pl.when(j >= NBUF - G)
            def _():
                s_wait(j - (NBUF - G))

            s_fire(j)

            @pl.when(j + G < NCH)
            def _():
                g_fire(j + G)

            def deg_body(i, _):
                d16 = dstv[j, pl.ds(i * 16, 16)]
                plsc.addupdate_scatter(degl, [d16], ones16)
                return 0
            lax.fori_loop(0, CHUNK // 16, deg_body, 0)
            return 0

        lax.fori_loop(0, NCH, chunk_body, 0)
        for j in range(NCH - (NBUF - G), NCH):
            s_wait(j)
        plsc.subcore_barrier()

        ob = sid * 624
        pltpu.sync_copy(acc.at[pl.ds(ob, 624)],
                        msum_out.at[cid, pl.ds(ob, 624)])

        @pl.when(sid == NS - 1)
        def _():
            pltpu.sync_copy(acc.at[pl.ds(16 * 624, N - 16 * 624)],
                            msum_out.at[cid, pl.ds(16 * 624, N - 16 * 624)])

        @pl.when(cid == 0)
        def _():
            pltpu.sync_copy(degl.at[pl.ds(0, N)], deg_out.at[sid, 0])

    return agg(h_cat, src2, dst_r)


def _tc_init(features, W_init, b_init):
    R = 2000

    def body(x_ref, w_ref, b_ref, o_ref):
        y = jnp.dot(x_ref[...], w_ref[...], preferred_element_type=jnp.float32)
        o_ref[...] = jnp.maximum(y + b_ref[...], 0.0)

    return pl.pallas_call(
        body,
        grid=(N // R,),
        in_specs=[
            pl.BlockSpec((R, IN_DIM), lambda i: (i, 0)),
            pl.BlockSpec((IN_DIM, HID), lambda i: (0, 0)),
            pl.BlockSpec((1, HID), lambda i: (0, 0)),
        ],
        out_specs=pl.BlockSpec((R, HID), lambda i: (i, 0)),
        out_shape=jax.ShapeDtypeStruct((N, HID), jnp.float32),
    )(features, W_init, b_init.reshape(1, HID))


def _tc_combine(h, msum, deg_t, W_self, b_self, W_neigh, b_neigh, act):
    R = 2000

    def body(h_ref, m_ref, d_ref, ws_ref, wn_ref, bs_ref, bn_ref, o_ref):
        h_blk = h_ref[...]
        msum_blk = jnp.concatenate([m_ref[0], m_ref[1]], axis=1)
        deg = jnp.sum(d_ref[...], axis=1)
        h_neigh = msum_blk / jnp.clip(deg, 1.0)[:, None]
        out = (jnp.dot(h_blk, ws_ref[...], preferred_element_type=jnp.float32)
               + jnp.dot(h_neigh, wn_ref[...],
                         preferred_element_type=jnp.float32)
               + bs_ref[...] + bn_ref[...])
        if act:
            out = jnp.maximum(out, 0.0)
        o_ref[...] = out

    return pl.pallas_call(
        body,
        grid=(N // R,),
        in_specs=[
            pl.BlockSpec((R, HID), lambda i: (i, 0)),
            pl.BlockSpec((NC, R, HH), lambda i: (0, i, 0)),
            pl.BlockSpec((R, NS), lambda i: (i, 0)),
            pl.BlockSpec((HID, HID), lambda i: (0, 0)),
            pl.BlockSpec((HID, HID), lambda i: (0, 0)),
            pl.BlockSpec((1, HID), lambda i: (0, 0)),
            pl.BlockSpec((1, HID), lambda i: (0, 0)),
        ],
        out_specs=pl.BlockSpec((R, HID), lambda i: (i, 0)),
        out_shape=jax.ShapeDtypeStruct((N, HID), jnp.float32),
    )(h, msum, deg_t, W_self, W_neigh,
      b_self.reshape(1, HID), b_neigh.reshape(1, HID))


def _prep_edges(edge_index):
    src = edge_index[0].astype(jnp.int32)
    dst = edge_index[1].astype(jnp.int32)
    pad = E_PAD - E
    src_p = jnp.concatenate([src, jnp.zeros((pad,), jnp.int32)])
    dst_p = jnp.concatenate([dst, jnp.full((pad,), N, jnp.int32)])
    src_r = src_p.reshape(NS, NCH, CHUNK)
    src2 = jnp.stack([src_r, src_r + N])
    return src2, dst_p.reshape(NS, NCH, CHUNK)


def _layer(h, src2, dst_r, W_self, b_self, W_neigh, b_neigh, act):
    h_cat = jnp.concatenate([h[:, :HH], h[:, HH:]], axis=0)
    msum, deg_p = _sc_aggregate(h_cat, src2, dst_r)
    return _tc_combine(h, msum, deg_p[:, 0, :].T, W_self, b_self,
                       W_neigh, b_neigh, act)


def kernel(features, edge_index0, edge_index1, W_init, b_init,
           W_self, b_self, W_neigh, b_neigh):
    src2_0, dst0 = _prep_edges(edge_index0)
    src2_1, dst1 = _prep_edges(edge_index1)

    h = _tc_init(features, W_init, b_init)
    h = _layer(h, src2_0, dst0, W_self, b_self, W_neigh, b_neigh, act=True)
    h = _layer(h, src2_1, dst1, W_self, b_self, W_neigh, b_neigh, act=False)
    return h
